# baseline (speedup 1.0000x reference)
import dataclasses
import functools

import jax
import jax.numpy as jnp
import numpy as np
from jax import lax
from jax.experimental import pallas as pl
from jax.experimental.pallas import tpu as pltpu
from jax.experimental.pallas import tpu_sc as plsc

VOCAB = 100000
DIM = 128
B = 4096
L = 50
HID = 600
NCLS = 5

NC = 2
NS = 16
NW = NC * NS
B_PER_W = B // NW
ROWS_PER_GATHER = 104
N_CHUNK = B_PER_W // 2
IDX_PER_W = N_CHUNK * ROWS_PER_GATHER
WORDS = DIM
NREG = 2 * DIM // 16

_PERM = np.empty(2 * DIM, np.int32)
for _j in range(2 * DIM // 32):
    for _k in range(16):
        _PERM[32 * _j + _k] = 32 * _j + 2 * _k
        _PERM[32 * _j + 16 + _k] = 32 * _j + 2 * _k + 1


def _sc_compiler_params():
    cp = pltpu.CompilerParams()
    if "needs_layout_passes" in pltpu.CompilerParams.__dataclass_fields__:
        cp = dataclasses.replace(cp, needs_layout_passes=False)
    return cp


def _emb_pool_sc(text_flat, fused_i32):
    mesh = plsc.VectorSubcoreMesh(core_axis_name="c", subcore_axis_name="s")

    @functools.partial(
        pl.kernel,
        compiler_params=_sc_compiler_params(),
        out_type=jax.ShapeDtypeStruct((B, 2 * DIM), jnp.float32),
        mesh=mesh,
        scratch_types=[
            pltpu.VMEM((IDX_PER_W,), jnp.int32),
            pltpu.VMEM((ROWS_PER_GATHER, WORDS), jnp.int32),
            pltpu.VMEM((ROWS_PER_GATHER, WORDS), jnp.int32),
            pltpu.VMEM((B_PER_W, 2 * DIM), jnp.float32),
            pltpu.SemaphoreType.DMA,
            pltpu.SemaphoreType.DMA,
        ],
    )
    def emb_kernel(text_hbm, table_hbm, out_hbm,
                   idx_v, rows0, rows1, out_v, sem0, sem1):
        wid = lax.axis_index("s") * NC + lax.axis_index("c")
        base_b = wid * B_PER_W
        pltpu.sync_copy(text_hbm.at[pl.ds(wid * IDX_PER_W, IDX_PER_W)], idx_v)

        def idx_slice(g):
            return idx_v.at[pl.ds(g * ROWS_PER_GATHER, ROWS_PER_GATHER)]

        hi_mask = jnp.full((16,), -65536, jnp.int32)

        def accumulate(rows_v, g):
            for bi in range(2):
                def body(l, carry):
                    out = list(carry)
                    for u in range(5):
                        row = bi * L + l * 5 + u
                        for j in range(WORDS // 16):
                            w = rows_v[row, pl.ds(j * 16, 16)]
                            a = plsc.bitcast(w << 16, jnp.float32)
                            b = plsc.bitcast(w & hi_mask, jnp.float32)
                            out[2 * j] = out[2 * j] + a
                            out[2 * j + 1] = out[2 * j + 1] + b
                    return tuple(out)

                acc = lax.fori_loop(
                    0, L // 5, body,
                    tuple(jnp.zeros((16,), jnp.float32) for _ in range(NREG)),
                )
                for k in range(NREG):
                    out_v[2 * g + bi, pl.ds(k * 16, 16)] = acc[k]

        ring = ((0, rows0, sem0), (1, rows1, sem1))
        NBUF = len(ring)

        for b, rows_v, sem in ring:
            pltpu.async_copy(table_hbm.at[idx_slice(b)], rows_v, sem)

        @pl.loop(0, N_CHUNK, step=NBUF)
        def _(g):
            for b, rows_v, sem in ring:
                gg = g + b
                pltpu.make_async_copy(
                    table_hbm.at[idx_slice(gg)], rows_v, sem).wait()
                accumulate(rows_v, gg)

                @pl.when(gg + NBUF < N_CHUNK)
                def _():
                    pltpu.async_copy(
                        table_hbm.at[idx_slice(gg + NBUF)], rows_v, sem)

        pltpu.sync_copy(out_v, out_hbm.at[pl.ds(base_b, B_PER_W)])

    return emb_kernel(text_flat, fused_i32)


def _mlp_body(e_ref, w1_ref, b1_ref, w2_ref, b2_ref, out_ref):
    h = jnp.dot(e_ref[...], w1_ref[...], preferred_element_type=jnp.float32,
                precision=lax.Precision.HIGHEST)
    h = jnp.maximum(h + b1_ref[...], 0.0)
    logits = jnp.dot(h, w2_ref[...], preferred_element_type=jnp.float32,
                     precision=lax.Precision.HIGHEST)
    logits = logits + b2_ref[...]
    m = jnp.max(logits, axis=-1, keepdims=True)
    s = logits - m
    lse = jnp.log(jnp.sum(jnp.exp(s), axis=-1, keepdims=True))
    out_ref[...] = s - lse


def _mlp_tc(emb, W1, b1, W2, b2):
    BLK = 512
    grid = (B // BLK,)
    return pl.pallas_call(
        _mlp_body,
        grid=grid,
        in_specs=[
            pl.BlockSpec((BLK, 2 * DIM), lambda i: (i, 0)),
            pl.BlockSpec((2 * DIM, HID), lambda i: (0, 0)),
            pl.BlockSpec((1, HID), lambda i: (0, 0)),
            pl.BlockSpec((HID, NCLS), lambda i: (0, 0)),
            pl.BlockSpec((1, NCLS), lambda i: (0, 0)),
        ],
        out_specs=pl.BlockSpec((BLK, NCLS), lambda i: (i, 0)),
        out_shape=jax.ShapeDtypeStruct((B, NCLS), jnp.float32),
    )(emb, W1, b1, W2, b2)


def _fused_i32_table(lut_w, static_w):
    fused = jnp.concatenate(
        [lut_w.astype(jnp.bfloat16), static_w.astype(jnp.bfloat16)], axis=1)
    return lax.bitcast_convert_type(
        fused.reshape(VOCAB, WORDS, 2), jnp.int32)


def kernel(text, lut_w, static_w, W1, b1, W2, b2):
    text2 = text.reshape(B // 2, 2 * L)
    text2 = jnp.pad(text2, ((0, 0), (0, ROWS_PER_GATHER - 2 * L)))
    text_flat = text2.reshape(B // 2 * ROWS_PER_GATHER)
    emb = _emb_pool_sc(text_flat, _fused_i32_table(lut_w, static_w))
    W1p = W1[_PERM, :]
    return _mlp_tc(emb, W1p, b1.reshape(1, HID), W2, b2.reshape(1, NCLS))


# speedup vs baseline: 1.8634x; 1.8634x over previous
import dataclasses
import functools

import jax
import jax.numpy as jnp
import numpy as np
from jax import lax
from jax.experimental import pallas as pl
from jax.experimental.pallas import tpu as pltpu
from jax.experimental.pallas import tpu_sc as plsc

VOCAB = 100000
DIM = 128
B = 4096
L = 50
HID = 600
NCLS = 5

NC = 2
NS = 16
NW = NC * NS
B_PER_W = B // NW
ROWS_PER_GATHER = 104
N_CHUNK = B_PER_W // 2
IDX_PER_W = N_CHUNK * ROWS_PER_GATHER
WORDS = DIM
NREG = 2 * DIM // 16

_PERM = np.empty(2 * DIM, np.int32)
for _jw in range(8):
    _base = 0 if _jw < 4 else DIM
    _o = (_jw % 4) * 16
    for _k in range(16):
        _PERM[32 * _jw + _k] = _base + _o + _k
        _PERM[32 * _jw + 16 + _k] = _base + 64 + _o + _k


def _sc_compiler_params():
    cp = pltpu.CompilerParams()
    if "needs_layout_passes" in pltpu.CompilerParams.__dataclass_fields__:
        cp = dataclasses.replace(cp, needs_layout_passes=False)
    return cp


def _emb_pool_sc(text_flat, fused_i32):
    mesh = plsc.VectorSubcoreMesh(core_axis_name="c", subcore_axis_name="s")

    @functools.partial(
        pl.kernel,
        compiler_params=_sc_compiler_params(),
        out_type=jax.ShapeDtypeStruct((B, 2 * DIM), jnp.float32),
        mesh=mesh,
        scratch_types=[
            pltpu.VMEM((IDX_PER_W,), jnp.int32),
            pltpu.VMEM((ROWS_PER_GATHER, WORDS), jnp.int32),
            pltpu.VMEM((ROWS_PER_GATHER, WORDS), jnp.int32),
            pltpu.VMEM((B_PER_W, 2 * DIM), jnp.float32),
            pltpu.SemaphoreType.DMA,
            pltpu.SemaphoreType.DMA,
        ],
    )
    def emb_kernel(text_hbm, table_hbm, out_hbm,
                   idx_v, rows0, rows1, out_v, sem0, sem1):
        wid = lax.axis_index("s") * NC + lax.axis_index("c")
        base_b = wid * B_PER_W
        pltpu.sync_copy(text_hbm.at[pl.ds(wid * IDX_PER_W, IDX_PER_W)], idx_v)

        def idx_slice(g):
            return idx_v.at[pl.ds(g * ROWS_PER_GATHER, ROWS_PER_GATHER)]

        hi_mask = jnp.full((16,), -65536, jnp.int32)

        def accumulate(rows_v, g):
            for bi in range(2):
                def body(l, carry):
                    out = list(carry)
                    for u in range(5):
                        row = bi * L + l * 5 + u
                        for j in range(WORDS // 16):
                            w = rows_v[row, pl.ds(j * 16, 16)]
                            a = plsc.bitcast(w << 16, jnp.float32)
                            b = plsc.bitcast(w & hi_mask, jnp.float32)
                            out[2 * j] = out[2 * j] + a
                            out[2 * j + 1] = out[2 * j + 1] + b
                    return tuple(out)

                acc = lax.fori_loop(
                    0, L // 5, body,
                    tuple(jnp.zeros((16,), jnp.float32) for _ in range(NREG)),
                )
                for k in range(NREG):
                    out_v[2 * g + bi, pl.ds(k * 16, 16)] = acc[k]

        ring = ((0, rows0, sem0), (1, rows1, sem1))
        NBUF = len(ring)

        for b, rows_v, sem in ring:
            pltpu.async_copy(table_hbm.at[idx_slice(b)], rows_v, sem)

        @pl.loop(0, N_CHUNK, step=NBUF)
        def _(g):
            for b, rows_v, sem in ring:
                gg = g + b
                pltpu.make_async_copy(
                    table_hbm.at[idx_slice(gg)], rows_v, sem).wait()
                accumulate(rows_v, gg)

                @pl.when(gg + NBUF < N_CHUNK)
                def _():
                    pltpu.async_copy(
                        table_hbm.at[idx_slice(gg + NBUF)], rows_v, sem)

        pltpu.sync_copy(out_v, out_hbm.at[pl.ds(base_b, B_PER_W)])

    return emb_kernel(text_flat, fused_i32)


def _mlp_body(e_ref, w1_ref, b1_ref, w2_ref, b2_ref, out_ref):
    h = jnp.dot(e_ref[...], w1_ref[...], preferred_element_type=jnp.float32,
                precision=lax.Precision.HIGHEST)
    h = jnp.maximum(h + b1_ref[...], 0.0)
    logits = jnp.dot(h, w2_ref[...], preferred_element_type=jnp.float32,
                     precision=lax.Precision.HIGHEST)
    logits = logits + b2_ref[...]
    m = jnp.max(logits, axis=-1, keepdims=True)
    s = logits - m
    lse = jnp.log(jnp.sum(jnp.exp(s), axis=-1, keepdims=True))
    out_ref[...] = s - lse


def _mlp_tc(emb, W1, b1, W2, b2):
    BLK = 512
    grid = (B // BLK,)
    return pl.pallas_call(
        _mlp_body,
        grid=grid,
        in_specs=[
            pl.BlockSpec((BLK, 2 * DIM), lambda i: (i, 0)),
            pl.BlockSpec((2 * DIM, HID), lambda i: (0, 0)),
            pl.BlockSpec((1, HID), lambda i: (0, 0)),
            pl.BlockSpec((HID, NCLS), lambda i: (0, 0)),
            pl.BlockSpec((1, NCLS), lambda i: (0, 0)),
        ],
        out_specs=pl.BlockSpec((BLK, NCLS), lambda i: (i, 0)),
        out_shape=jax.ShapeDtypeStruct((B, NCLS), jnp.float32),
    )(emb, W1, b1, W2, b2)


def _pack_body(lut_ref, static_ref, out_ref):
    for half, src in ((0, lut_ref), (1, static_ref)):
        u = lax.bitcast_convert_type(
            src[...].astype(jnp.bfloat16), jnp.uint16).astype(jnp.uint32)
        w = (u[:, 64:] << 16) | u[:, :64]
        out_ref[:, half * 64:(half + 1) * 64] = lax.bitcast_convert_type(
            w, jnp.int32)


def _fused_i32_table(lut_w, static_w):
    VB = 1000
    return pl.pallas_call(
        _pack_body,
        grid=(VOCAB // VB,),
        in_specs=[
            pl.BlockSpec((VB, DIM), lambda i: (i, 0)),
            pl.BlockSpec((VB, DIM), lambda i: (i, 0)),
        ],
        out_specs=pl.BlockSpec((VB, WORDS), lambda i: (i, 0)),
        out_shape=jax.ShapeDtypeStruct((VOCAB, WORDS), jnp.int32),
    )(lut_w, static_w)


def kernel(text, lut_w, static_w, W1, b1, W2, b2):
    text2 = text.reshape(B // 2, 2 * L)
    text2 = jnp.pad(text2, ((0, 0), (0, ROWS_PER_GATHER - 2 * L)))
    text_flat = text2.reshape(B // 2 * ROWS_PER_GATHER)
    emb = _emb_pool_sc(text_flat, _fused_i32_table(lut_w, static_w))
    W1p = W1[_PERM, :]
    return _mlp_tc(emb, W1p, b1.reshape(1, HID), W2, b2.reshape(1, NCLS))


# speedup vs baseline: 1.9566x; 1.0500x over previous
import dataclasses
import functools

import jax
import jax.numpy as jnp
import numpy as np
from jax import lax
from jax.experimental import pallas as pl
from jax.experimental.pallas import tpu as pltpu
from jax.experimental.pallas import tpu_sc as plsc

VOCAB = 100000
DIM = 128
B = 4096
L = 50
HID = 600
NCLS = 5

NC = 2
NS = 16
NW = NC * NS
B_PER_W = B // NW
ROWS_PER_GATHER = 104
N_CHUNK = B_PER_W // 2
IDX_PER_W = N_CHUNK * ROWS_PER_GATHER
WORDS = DIM
NREG = 2 * DIM // 16

_PERM = np.empty(2 * DIM, np.int32)
for _jw in range(8):
    _base = 0 if _jw < 4 else DIM
    _o = (_jw % 4) * 16
    for _k in range(16):
        _PERM[32 * _jw + _k] = _base + _o + _k
        _PERM[32 * _jw + 16 + _k] = _base + 64 + _o + _k


def _sc_compiler_params():
    cp = pltpu.CompilerParams()
    if "needs_layout_passes" in pltpu.CompilerParams.__dataclass_fields__:
        cp = dataclasses.replace(cp, needs_layout_passes=False)
    return cp


def _emb_pool_sc(text_flat, fused_i32):
    mesh = plsc.VectorSubcoreMesh(core_axis_name="c", subcore_axis_name="s")

    @functools.partial(
        pl.kernel,
        compiler_params=_sc_compiler_params(),
        out_type=jax.ShapeDtypeStruct((B, 2 * DIM), jnp.float32),
        mesh=mesh,
        scratch_types=[
            pltpu.VMEM((IDX_PER_W,), jnp.int32),
            pltpu.VMEM((ROWS_PER_GATHER, WORDS), jnp.int32),
            pltpu.VMEM((ROWS_PER_GATHER, WORDS), jnp.int32),
            pltpu.VMEM((B_PER_W, 2 * DIM), jnp.float32),
            pltpu.SemaphoreType.DMA,
            pltpu.SemaphoreType.DMA,
        ],
    )
    def emb_kernel(text_hbm, table_hbm, out_hbm,
                   idx_v, rows0, rows1, out_v, sem0, sem1):
        wid = lax.axis_index("s") * NC + lax.axis_index("c")
        base_b = wid * B_PER_W
        pltpu.sync_copy(text_hbm.at[pl.ds(wid * IDX_PER_W, IDX_PER_W)], idx_v)

        def idx_slice(g):
            return idx_v.at[pl.ds(g * ROWS_PER_GATHER, ROWS_PER_GATHER)]

        hi_mask = jnp.full((16,), -65536, jnp.int32)

        def accumulate(rows_v, g):
            for bi in range(2):
                def body(l, carry):
                    out = list(carry)
                    for u in range(5):
                        row = bi * L + l * 5 + u
                        for j in range(WORDS // 16):
                            w = rows_v[row, pl.ds(j * 16, 16)]
                            a = plsc.bitcast(w << 16, jnp.float32)
                            b = plsc.bitcast(w & hi_mask, jnp.float32)
                            out[2 * j] = out[2 * j] + a
                            out[2 * j + 1] = out[2 * j + 1] + b
                    return tuple(out)

                acc = lax.fori_loop(
                    0, L // 5, body,
                    tuple(jnp.zeros((16,), jnp.float32) for _ in range(NREG)),
                )
                for k in range(NREG):
                    out_v[2 * g + bi, pl.ds(k * 16, 16)] = acc[k]

        ring = ((0, rows0, sem0), (1, rows1, sem1))
        NBUF = len(ring)

        for b, rows_v, sem in ring:
            pltpu.async_copy(table_hbm.at[idx_slice(b)], rows_v, sem)

        @pl.loop(0, N_CHUNK, step=NBUF)
        def _(g):
            for b, rows_v, sem in ring:
                gg = g + b
                pltpu.make_async_copy(
                    table_hbm.at[idx_slice(gg)], rows_v, sem).wait()
                accumulate(rows_v, gg)

                @pl.when(gg + NBUF < N_CHUNK)
                def _():
                    pltpu.async_copy(
                        table_hbm.at[idx_slice(gg + NBUF)], rows_v, sem)

        pltpu.sync_copy(out_v, out_hbm.at[pl.ds(base_b, B_PER_W)])

    return emb_kernel(text_flat, fused_i32)


def _mlp_body(e_ref, w1_ref, b1_ref, w2_ref, b2_ref, out_ref):
    h = jnp.dot(e_ref[...], w1_ref[...], preferred_element_type=jnp.float32,
                precision=lax.Precision.HIGHEST)
    h = jnp.maximum(h + b1_ref[...], 0.0)
    logits = jnp.dot(h, w2_ref[...], preferred_element_type=jnp.float32,
                     precision=lax.Precision.HIGHEST)
    logits = logits + b2_ref[...]
    m = jnp.max(logits, axis=-1, keepdims=True)
    s = logits - m
    lse = jnp.log(jnp.sum(jnp.exp(s), axis=-1, keepdims=True))
    out_ref[...] = s - lse


def _mlp_tc(emb, W1, b1, W2, b2):
    BLK = 512
    grid = (B // BLK,)
    return pl.pallas_call(
        _mlp_body,
        grid=grid,
        in_specs=[
            pl.BlockSpec((BLK, 2 * DIM), lambda i: (i, 0)),
            pl.BlockSpec((2 * DIM, HID), lambda i: (0, 0)),
            pl.BlockSpec((1, HID), lambda i: (0, 0)),
            pl.BlockSpec((HID, NCLS), lambda i: (0, 0)),
            pl.BlockSpec((1, NCLS), lambda i: (0, 0)),
        ],
        out_specs=pl.BlockSpec((BLK, NCLS), lambda i: (i, 0)),
        out_shape=jax.ShapeDtypeStruct((B, NCLS), jnp.float32),
    )(emb, W1, b1, W2, b2)


def _pack_body(lut_ref, static_ref, out_ref):
    def rne_lo(x):
        b = lax.bitcast_convert_type(x, jnp.uint32)
        return (b + jnp.uint32(0x7FFF) + ((b >> 16) & jnp.uint32(1))) >> 16

    def rne_hi(x):
        b = lax.bitcast_convert_type(x, jnp.uint32)
        return (b + jnp.uint32(0x7FFF) + ((b >> 16) & jnp.uint32(1))) & (
            jnp.uint32(0xFFFF0000))

    for half, src in ((0, lut_ref), (1, static_ref)):
        x = src[...]
        w = rne_lo(x[:, :64]) | rne_hi(x[:, 64:])
        out_ref[:, half * 64:(half + 1) * 64] = lax.bitcast_convert_type(
            w, jnp.int32)


def _fused_i32_table(lut_w, static_w):
    VB = 2000
    return pl.pallas_call(
        _pack_body,
        grid=(VOCAB // VB,),
        in_specs=[
            pl.BlockSpec((VB, DIM), lambda i: (i, 0)),
            pl.BlockSpec((VB, DIM), lambda i: (i, 0)),
        ],
        out_specs=pl.BlockSpec((VB, WORDS), lambda i: (i, 0)),
        out_shape=jax.ShapeDtypeStruct((VOCAB, WORDS), jnp.int32),
    )(lut_w, static_w)


def kernel(text, lut_w, static_w, W1, b1, W2, b2):
    text2 = text.reshape(B // 2, 2 * L)
    text2 = jnp.pad(text2, ((0, 0), (0, ROWS_PER_GATHER - 2 * L)))
    text_flat = text2.reshape(B // 2 * ROWS_PER_GATHER)
    emb = _emb_pool_sc(text_flat, _fused_i32_table(lut_w, static_w))
    W1p = W1[_PERM, :]
    return _mlp_tc(emb, W1p, b1.reshape(1, HID), W2, b2.reshape(1, NCLS))


# speedup vs baseline: 1.9851x; 1.0146x over previous
import dataclasses
import functools

import jax
import jax.numpy as jnp
import numpy as np
from jax import lax
from jax.experimental import pallas as pl
from jax.experimental.pallas import tpu as pltpu
from jax.experimental.pallas import tpu_sc as plsc

VOCAB = 100000
DIM = 128
B = 4096
L = 50
HID = 600
NCLS = 5

NC = 2
NS = 16
NW = NC * NS
B_PER_W = B // NW
ROWS_PER_GATHER = 104
N_CHUNK = B_PER_W // 2
IDX_PER_W = N_CHUNK * ROWS_PER_GATHER
WORDS = DIM
NREG = 2 * DIM // 16

_PERM = np.empty(2 * DIM, np.int32)
for _jw in range(8):
    _base = 0 if _jw < 4 else DIM
    _o = (_jw % 4) * 16
    for _k in range(16):
        _PERM[32 * _jw + _k] = _base + _o + _k
        _PERM[32 * _jw + 16 + _k] = _base + 64 + _o + _k


def _sc_compiler_params():
    cp = pltpu.CompilerParams()
    if "needs_layout_passes" in pltpu.CompilerParams.__dataclass_fields__:
        cp = dataclasses.replace(cp, needs_layout_passes=False)
    return cp


def _emb_pool_sc(text_flat, fused_i32, nb):
    mesh = plsc.VectorSubcoreMesh(core_axis_name="c", subcore_axis_name="s")
    b_per_w = nb // NW
    n_chunk = b_per_w // 2
    idx_per_w = n_chunk * ROWS_PER_GATHER

    @functools.partial(
        pl.kernel,
        compiler_params=_sc_compiler_params(),
        out_type=jax.ShapeDtypeStruct((nb, 2 * DIM), jnp.float32),
        mesh=mesh,
        scratch_types=[
            pltpu.VMEM((idx_per_w,), jnp.int32),
            pltpu.VMEM((ROWS_PER_GATHER, WORDS), jnp.int32),
            pltpu.VMEM((ROWS_PER_GATHER, WORDS), jnp.int32),
            pltpu.VMEM((b_per_w, 2 * DIM), jnp.float32),
            pltpu.SemaphoreType.DMA,
            pltpu.SemaphoreType.DMA,
        ],
    )
    def emb_kernel(text_hbm, table_hbm, out_hbm,
                   idx_v, rows0, rows1, out_v, sem0, sem1):
        wid = lax.axis_index("s") * NC + lax.axis_index("c")
        base_b = wid * b_per_w
        pltpu.sync_copy(text_hbm.at[pl.ds(wid * idx_per_w, idx_per_w)], idx_v)

        def idx_slice(g):
            return idx_v.at[pl.ds(g * ROWS_PER_GATHER, ROWS_PER_GATHER)]

        hi_mask = jnp.full((16,), -65536, jnp.int32)

        def accumulate(rows_v, g):
            for bi in range(2):
                def body(l, carry):
                    out = list(carry)
                    for u in range(5):
                        row = bi * L + l * 5 + u
                        for j in range(WORDS // 16):
                            w = rows_v[row, pl.ds(j * 16, 16)]
                            a = plsc.bitcast(w << 16, jnp.float32)
                            b = plsc.bitcast(w & hi_mask, jnp.float32)
                            out[2 * j] = out[2 * j] + a
                            out[2 * j + 1] = out[2 * j + 1] + b
                    return tuple(out)

                acc = lax.fori_loop(
                    0, L // 5, body,
                    tuple(jnp.zeros((16,), jnp.float32) for _ in range(NREG)),
                )
                for k in range(NREG):
                    out_v[2 * g + bi, pl.ds(k * 16, 16)] = acc[k]

        ring = ((0, rows0, sem0), (1, rows1, sem1))
        NBUF = len(ring)

        for b, rows_v, sem in ring:
            pltpu.async_copy(table_hbm.at[idx_slice(b)], rows_v, sem)

        @pl.loop(0, n_chunk, step=NBUF)
        def _(g):
            for b, rows_v, sem in ring:
                gg = g + b
                pltpu.make_async_copy(
                    table_hbm.at[idx_slice(gg)], rows_v, sem).wait()
                accumulate(rows_v, gg)

                @pl.when(gg + NBUF < n_chunk)
                def _():
                    pltpu.async_copy(
                        table_hbm.at[idx_slice(gg + NBUF)], rows_v, sem)

        pltpu.sync_copy(out_v, out_hbm.at[pl.ds(base_b, b_per_w)])

    return emb_kernel(text_flat, fused_i32)


def _mlp_body(e_ref, w1_ref, b1_ref, w2_ref, b2_ref, out_ref):
    h = jnp.dot(e_ref[...], w1_ref[...], preferred_element_type=jnp.float32,
                precision=lax.Precision.HIGHEST)
    h = jnp.maximum(h + b1_ref[...], 0.0)
    logits = jnp.dot(h, w2_ref[...], preferred_element_type=jnp.float32,
                     precision=lax.Precision.HIGHEST)
    logits = logits + b2_ref[...]
    m = jnp.max(logits, axis=-1, keepdims=True)
    s = logits - m
    lse = jnp.log(jnp.sum(jnp.exp(s), axis=-1, keepdims=True))
    out_ref[...] = s - lse


def _mlp_tc(emb, W1, b1, W2, b2):
    BLK = 512
    nb = emb.shape[0]
    grid = (nb // BLK,)
    return pl.pallas_call(
        _mlp_body,
        grid=grid,
        in_specs=[
            pl.BlockSpec((BLK, 2 * DIM), lambda i: (i, 0)),
            pl.BlockSpec((2 * DIM, HID), lambda i: (0, 0)),
            pl.BlockSpec((1, HID), lambda i: (0, 0)),
            pl.BlockSpec((HID, NCLS), lambda i: (0, 0)),
            pl.BlockSpec((1, NCLS), lambda i: (0, 0)),
        ],
        out_specs=pl.BlockSpec((BLK, NCLS), lambda i: (i, 0)),
        out_shape=jax.ShapeDtypeStruct((nb, NCLS), jnp.float32),
    )(emb, W1, b1, W2, b2)


def _pack_body(lut_ref, static_ref, out_ref):
    def rne_lo(x):
        b = lax.bitcast_convert_type(x, jnp.uint32)
        return (b + jnp.uint32(0x7FFF) + ((b >> 16) & jnp.uint32(1))) >> 16

    def rne_hi(x):
        b = lax.bitcast_convert_type(x, jnp.uint32)
        return (b + jnp.uint32(0x7FFF) + ((b >> 16) & jnp.uint32(1))) & (
            jnp.uint32(0xFFFF0000))

    for half, src in ((0, lut_ref), (1, static_ref)):
        x = src[...]
        w = rne_lo(x[:, :64]) | rne_hi(x[:, 64:])
        out_ref[:, half * 64:(half + 1) * 64] = lax.bitcast_convert_type(
            w, jnp.int32)


def _fused_i32_table(lut_w, static_w):
    VB = 2000
    return pl.pallas_call(
        _pack_body,
        grid=(VOCAB // VB,),
        in_specs=[
            pl.BlockSpec((VB, DIM), lambda i: (i, 0)),
            pl.BlockSpec((VB, DIM), lambda i: (i, 0)),
        ],
        out_specs=pl.BlockSpec((VB, WORDS), lambda i: (i, 0)),
        out_shape=jax.ShapeDtypeStruct((VOCAB, WORDS), jnp.int32),
    )(lut_w, static_w)


def kernel(text, lut_w, static_w, W1, b1, W2, b2):
    text2 = text.reshape(B // 2, 2 * L)
    text2 = jnp.pad(text2, ((0, 0), (0, ROWS_PER_GATHER - 2 * L)))
    text_flat = text2.reshape(B // 2 * ROWS_PER_GATHER)
    fused = _fused_i32_table(lut_w, static_w)
    W1p = W1[_PERM, :]
    b1r, b2r = b1.reshape(1, HID), b2.reshape(1, NCLS)
    half = text_flat.shape[0] // 2
    outs = []
    for h in range(2):
        emb = _emb_pool_sc(
            lax.dynamic_slice_in_dim(text_flat, h * half, half),
            fused, B // 2)
        outs.append(_mlp_tc(emb, W1p, b1r, W2, b2r))
    return jnp.concatenate(outs, axis=0)


# speedup vs baseline: 2.0874x; 1.0516x over previous
import dataclasses
import functools

import jax
import jax.numpy as jnp
import numpy as np
from jax import lax
from jax.experimental import pallas as pl
from jax.experimental.pallas import tpu as pltpu
from jax.experimental.pallas import tpu_sc as plsc

VOCAB = 100000
DIM = 128
B = 4096
L = 50
HID = 600
NCLS = 5

NC = 2
NS = 16
NW = NC * NS
B_PER_W = B // NW
ROWS_PER_GATHER = 104
N_CHUNK = B_PER_W // 2
IDX_PER_W = N_CHUNK * ROWS_PER_GATHER
WORDS = DIM
NREG = 2 * DIM // 16

_PERM = np.empty(2 * DIM, np.int32)
for _jw in range(8):
    _base = 0 if _jw < 4 else DIM
    _o = (_jw % 4) * 16
    for _k in range(16):
        _PERM[32 * _jw + _k] = _base + _o + _k
        _PERM[32 * _jw + 16 + _k] = _base + 64 + _o + _k


def _sc_compiler_params():
    cp = pltpu.CompilerParams()
    if "needs_layout_passes" in pltpu.CompilerParams.__dataclass_fields__:
        cp = dataclasses.replace(cp, needs_layout_passes=False)
    return cp


def _emb_pool_sc(text_flat, fused_i32, nb):
    mesh = plsc.VectorSubcoreMesh(core_axis_name="c", subcore_axis_name="s")
    b_per_w = nb // NW
    n_chunk = b_per_w // 2
    idx_per_w = n_chunk * ROWS_PER_GATHER

    @functools.partial(
        pl.kernel,
        compiler_params=_sc_compiler_params(),
        out_type=jax.ShapeDtypeStruct((nb, 2 * DIM), jnp.float32),
        mesh=mesh,
        scratch_types=[
            pltpu.VMEM((idx_per_w,), jnp.int32),
            pltpu.VMEM((ROWS_PER_GATHER, WORDS), jnp.int32),
            pltpu.VMEM((ROWS_PER_GATHER, WORDS), jnp.int32),
            pltpu.VMEM((b_per_w, 2 * DIM), jnp.float32),
            pltpu.SemaphoreType.DMA,
            pltpu.SemaphoreType.DMA,
        ],
    )
    def emb_kernel(text_hbm, table_hbm, out_hbm,
                   idx_v, rows0, rows1, out_v, sem0, sem1):
        wid = lax.axis_index("s") * NC + lax.axis_index("c")
        base_b = wid * b_per_w
        pltpu.sync_copy(text_hbm.at[pl.ds(wid * idx_per_w, idx_per_w)], idx_v)

        def idx_slice(g):
            return idx_v.at[pl.ds(g * ROWS_PER_GATHER, ROWS_PER_GATHER)]

        hi_mask = jnp.full((16,), -65536, jnp.int32)

        def accumulate(rows_v, g):
            for bi in range(2):
                def body(l, carry):
                    out = list(carry)
                    for u in range(5):
                        row = bi * L + l * 5 + u
                        for j in range(WORDS // 16):
                            w = rows_v[row, pl.ds(j * 16, 16)]
                            a = plsc.bitcast(w << 16, jnp.float32)
                            b = plsc.bitcast(w & hi_mask, jnp.float32)
                            out[2 * j] = out[2 * j] + a
                            out[2 * j + 1] = out[2 * j + 1] + b
                    return tuple(out)

                acc = lax.fori_loop(
                    0, L // 5, body,
                    tuple(jnp.zeros((16,), jnp.float32) for _ in range(NREG)),
                )
                for k in range(NREG):
                    out_v[2 * g + bi, pl.ds(k * 16, 16)] = acc[k]

        ring = ((0, rows0, sem0), (1, rows1, sem1))
        NBUF = len(ring)

        for b, rows_v, sem in ring:
            pltpu.async_copy(table_hbm.at[idx_slice(b)], rows_v, sem)

        @pl.loop(0, n_chunk, step=NBUF)
        def _(g):
            for b, rows_v, sem in ring:
                gg = g + b
                pltpu.make_async_copy(
                    table_hbm.at[idx_slice(gg)], rows_v, sem).wait()
                accumulate(rows_v, gg)

                @pl.when(gg + NBUF < n_chunk)
                def _():
                    pltpu.async_copy(
                        table_hbm.at[idx_slice(gg + NBUF)], rows_v, sem)

        pltpu.sync_copy(out_v, out_hbm.at[pl.ds(base_b, b_per_w)])

    return emb_kernel(text_flat, fused_i32)


def _mlp_body(e_ref, w1_ref, b1_ref, w2_ref, b2_ref, out_ref):
    h = jnp.dot(e_ref[...], w1_ref[...], preferred_element_type=jnp.float32)
    h = jnp.maximum(h + b1_ref[...], 0.0)
    logits = jnp.dot(h, w2_ref[...], preferred_element_type=jnp.float32)
    logits = logits + b2_ref[...]
    m = jnp.max(logits, axis=-1, keepdims=True)
    s = logits - m
    lse = jnp.log(jnp.sum(jnp.exp(s), axis=-1, keepdims=True))
    out_ref[...] = s - lse


def _mlp_tc(emb, W1, b1, W2, b2):
    BLK = 512
    nb = emb.shape[0]
    grid = (nb // BLK,)
    return pl.pallas_call(
        _mlp_body,
        grid=grid,
        in_specs=[
            pl.BlockSpec((BLK, 2 * DIM), lambda i: (i, 0)),
            pl.BlockSpec((2 * DIM, HID), lambda i: (0, 0)),
            pl.BlockSpec((1, HID), lambda i: (0, 0)),
            pl.BlockSpec((HID, NCLS), lambda i: (0, 0)),
            pl.BlockSpec((1, NCLS), lambda i: (0, 0)),
        ],
        out_specs=pl.BlockSpec((BLK, NCLS), lambda i: (i, 0)),
        out_shape=jax.ShapeDtypeStruct((nb, NCLS), jnp.float32),
    )(emb, W1, b1, W2, b2)


def _pack_body(lut_ref, static_ref, out_ref):
    def rne_lo(x):
        b = lax.bitcast_convert_type(x, jnp.uint32)
        return (b + jnp.uint32(0x7FFF) + ((b >> 16) & jnp.uint32(1))) >> 16

    def rne_hi(x):
        b = lax.bitcast_convert_type(x, jnp.uint32)
        return (b + jnp.uint32(0x7FFF) + ((b >> 16) & jnp.uint32(1))) & (
            jnp.uint32(0xFFFF0000))

    for half, src in ((0, lut_ref), (1, static_ref)):
        x = src[...]
        w = rne_lo(x[:, :64]) | rne_hi(x[:, 64:])
        out_ref[:, half * 64:(half + 1) * 64] = lax.bitcast_convert_type(
            w, jnp.int32)


def _fused_i32_table(lut_w, static_w):
    VB = 4000
    return pl.pallas_call(
        _pack_body,
        grid=(VOCAB // VB,),
        in_specs=[
            pl.BlockSpec((VB, DIM), lambda i: (i, 0)),
            pl.BlockSpec((VB, DIM), lambda i: (i, 0)),
        ],
        out_specs=pl.BlockSpec((VB, WORDS), lambda i: (i, 0)),
        out_shape=jax.ShapeDtypeStruct((VOCAB, WORDS), jnp.int32),
    )(lut_w, static_w)


def kernel(text, lut_w, static_w, W1, b1, W2, b2):
    text2 = text.reshape(B // 2, 2 * L)
    text2 = jnp.pad(text2, ((0, 0), (0, ROWS_PER_GATHER - 2 * L)))
    text_flat = text2.reshape(B // 2 * ROWS_PER_GATHER)
    fused = _fused_i32_table(lut_w, static_w)
    W1p = W1[_PERM, :]
    b1r, b2r = b1.reshape(1, HID), b2.reshape(1, NCLS)
    half = text_flat.shape[0] // 2
    outs = []
    for h in range(2):
        emb = _emb_pool_sc(
            lax.dynamic_slice_in_dim(text_flat, h * half, half),
            fused, B // 2)
        outs.append(_mlp_tc(emb, W1p, b1r, W2, b2r))
    return jnp.concatenate(outs, axis=0)
